# Initial kernel scaffold; baseline (speedup 1.0000x reference)
import functools

import jax
import jax.numpy as jnp
from jax import lax
from jax.experimental import pallas as pl
from jax.experimental.pallas import tpu as pltpu
from jax.experimental.pallas import tpu_sc as plsc

N_NODES = 10000
D = 128
NC = 2
NS = 16
NW = NC * NS
EPB = 128
N_ACC = 10240
CHUNK = N_ACC // NS
DEG_W = 16
ROWS = 400
_MESH = dict(core_axis_name="c", subcore_axis_name="s")


def _deg_call(n_blk, dst_r, ones_hbm, zeros_hbm):
    @functools.partial(
        pl.kernel,
        out_type=jax.ShapeDtypeStruct((NC, N_ACC, DEG_W), jnp.float32),
        mesh=plsc.VectorSubcoreMesh(**_MESH),
        scratch_types=[
            pltpu.VMEM((n_blk, EPB), jnp.int32),
            pltpu.VMEM((EPB, DEG_W), jnp.float32),
            pltpu.VMEM_SHARED((N_ACC, DEG_W), jnp.float32),
        ],
    )
    def deg_kernel(dst_hbm, ones_h, zeros_h, deg_hbm, idx_v, ones_v, acc_sh):
        c = lax.axis_index("c")
        s = lax.axis_index("s")
        wid = c * NS + s
        pltpu.sync_copy(zeros_h.at[pl.ds(s * CHUNK, CHUNK)],
                        acc_sh.at[pl.ds(s * CHUNK, CHUNK)])
        pltpu.sync_copy(dst_hbm.at[wid], idx_v)
        pltpu.sync_copy(ones_h, ones_v)
        plsc.subcore_barrier()

        @pl.loop(0, n_blk)
        def _(j):
            pltpu.sync_copy(ones_v, acc_sh.at[idx_v.at[j]], add=True)

        plsc.subcore_barrier()
        pltpu.sync_copy(acc_sh.at[pl.ds(s * CHUNK, CHUNK)],
                        deg_hbm.at[c].at[pl.ds(s * CHUNK, CHUNK)])

    return deg_kernel(dst_r, ones_hbm, zeros_hbm)


def _scatter_call(n_blk, y, src_r, dst_r, zeros_hbm):
    @functools.partial(
        pl.kernel,
        out_type=jax.ShapeDtypeStruct((NC, N_ACC, D), jnp.float32),
        mesh=plsc.VectorSubcoreMesh(**_MESH),
        scratch_types=[
            pltpu.VMEM((n_blk, EPB), jnp.int32),
            pltpu.VMEM((n_blk, EPB), jnp.int32),
            pltpu.VMEM((EPB, D), jnp.float32),
            pltpu.VMEM((EPB, D), jnp.float32),
            pltpu.VMEM_SHARED((N_ACC, D), jnp.float32),
            pltpu.SemaphoreType.DMA,
            pltpu.SemaphoreType.DMA,
        ],
    )
    def scat_kernel(y_hbm, src_hbm, dst_hbm, zeros_h, out_hbm,
                    si, di, g0, g1, acc_sh, sem0, sem1):
        c = lax.axis_index("c")
        s = lax.axis_index("s")
        wid = c * NS + s
        pltpu.sync_copy(zeros_h.at[pl.ds(s * CHUNK, CHUNK)],
                        acc_sh.at[pl.ds(s * CHUNK, CHUNK)])
        pltpu.sync_copy(src_hbm.at[wid], si)
        pltpu.sync_copy(dst_hbm.at[wid], di)
        plsc.subcore_barrier()

        pltpu.async_copy(y_hbm.at[si.at[0]], g0, sem0)

        @pl.loop(0, n_blk - 2, step=2)
        def _(j):
            pltpu.async_copy(y_hbm.at[si.at[j + 1]], g1, sem1)
            pltpu.make_async_copy(y_hbm.at[si.at[j]], g0, sem0).wait()
            pltpu.sync_copy(g0, acc_sh.at[di.at[j]], add=True)
            pltpu.async_copy(y_hbm.at[si.at[j + 2]], g0, sem0)
            pltpu.make_async_copy(y_hbm.at[si.at[j + 1]], g1, sem1).wait()
            pltpu.sync_copy(g1, acc_sh.at[di.at[j + 1]], add=True)

        pltpu.async_copy(y_hbm.at[si.at[n_blk - 1]], g1, sem1)
        pltpu.make_async_copy(y_hbm.at[si.at[n_blk - 2]], g0, sem0).wait()
        pltpu.sync_copy(g0, acc_sh.at[di.at[n_blk - 2]], add=True)
        pltpu.make_async_copy(y_hbm.at[si.at[n_blk - 1]], g1, sem1).wait()
        pltpu.sync_copy(g1, acc_sh.at[di.at[n_blk - 1]], add=True)

        plsc.subcore_barrier()
        pltpu.sync_copy(acc_sh.at[pl.ds(s * CHUNK, CHUNK)],
                        out_hbm.at[c].at[pl.ds(s * CHUNK, CHUNK)])

    return scat_kernel(y, src_r, dst_r, zeros_hbm)


def _pre_call(x, W1, deg2):
    def body(x_ref, w_ref, deg_ref, y_ref, dinv_ref):
        d = deg_ref[0, :, 0:1] + deg_ref[1, :, 0:1] + 1.0
        dinv = lax.rsqrt(d)
        xw = jnp.dot(x_ref[...], w_ref[...], preferred_element_type=jnp.float32)
        y_ref[...] = xw * dinv
        dinv_ref[...] = jnp.broadcast_to(dinv, (ROWS, D))

    return pl.pallas_call(
        body,
        grid=(N_NODES // ROWS,),
        in_specs=[
            pl.BlockSpec((ROWS, D), lambda i: (i, 0)),
            pl.BlockSpec((D, D), lambda i: (0, 0)),
            pl.BlockSpec((NC, ROWS, DEG_W), lambda i: (0, i, 0)),
        ],
        out_specs=[
            pl.BlockSpec((ROWS, D), lambda i: (i, 0)),
            pl.BlockSpec((ROWS, D), lambda i: (i, 0)),
        ],
        out_shape=[
            jax.ShapeDtypeStruct((N_NODES, D), jnp.float32),
            jax.ShapeDtypeStruct((N_NODES, D), jnp.float32),
        ],
    )(x, W1, deg2)


def _mid_call(acc, y, dinv, b, h_prev, W_next):
    def body(acc_ref, y_ref, dinv_ref, b_ref, hp_ref, w_ref, h_ref, yn_ref):
        conv = dinv_ref[...] * (acc_ref[0] + acc_ref[1] + y_ref[...]) + b_ref[...]
        h = jnp.maximum(conv + hp_ref[...], 0.0)
        h_ref[...] = h
        hw = jnp.dot(h, w_ref[...], preferred_element_type=jnp.float32)
        yn_ref[...] = hw * dinv_ref[...]

    return pl.pallas_call(
        body,
        grid=(N_NODES // ROWS,),
        in_specs=[
            pl.BlockSpec((NC, ROWS, D), lambda i: (0, i, 0)),
            pl.BlockSpec((ROWS, D), lambda i: (i, 0)),
            pl.BlockSpec((ROWS, D), lambda i: (i, 0)),
            pl.BlockSpec((1, D), lambda i: (0, 0)),
            pl.BlockSpec((ROWS, D), lambda i: (i, 0)),
            pl.BlockSpec((D, D), lambda i: (0, 0)),
        ],
        out_specs=[
            pl.BlockSpec((ROWS, D), lambda i: (i, 0)),
            pl.BlockSpec((ROWS, D), lambda i: (i, 0)),
        ],
        out_shape=[
            jax.ShapeDtypeStruct((N_NODES, D), jnp.float32),
            jax.ShapeDtypeStruct((N_NODES, D), jnp.float32),
        ],
    )(acc, y, dinv, b, h_prev, W_next)


def _final_call(acc, y, dinv, b, h_prev):
    def body(acc_ref, y_ref, dinv_ref, b_ref, hp_ref, h_ref):
        conv = dinv_ref[...] * (acc_ref[0] + acc_ref[1] + y_ref[...]) + b_ref[...]
        h_ref[...] = jnp.maximum(conv + hp_ref[...], 0.0)

    return pl.pallas_call(
        body,
        grid=(N_NODES // ROWS,),
        in_specs=[
            pl.BlockSpec((NC, ROWS, D), lambda i: (0, i, 0)),
            pl.BlockSpec((ROWS, D), lambda i: (i, 0)),
            pl.BlockSpec((ROWS, D), lambda i: (i, 0)),
            pl.BlockSpec((1, D), lambda i: (0, 0)),
            pl.BlockSpec((ROWS, D), lambda i: (i, 0)),
        ],
        out_specs=pl.BlockSpec((ROWS, D), lambda i: (i, 0)),
        out_shape=jax.ShapeDtypeStruct((N_NODES, D), jnp.float32),
    )(acc, y, dinv, b, h_prev)


def kernel(x, edge_index, W1, b1, W2, b2, W3, b3):
    ei = edge_index.astype(jnp.int32)
    src, dst = ei[0], ei[1]
    e = src.shape[0]
    stride = NW * EPB
    n_blk = -(-e // stride)
    if n_blk % 2:
        n_blk += 1
    e_pad = n_blk * stride
    pad = e_pad - e
    src_p = jnp.concatenate([src, jnp.zeros((pad,), jnp.int32)])
    dst_p = jnp.concatenate([dst, jnp.full((pad,), N_NODES, jnp.int32)])
    src_r = src_p.reshape(NW, n_blk, EPB)
    dst_r = dst_p.reshape(NW, n_blk, EPB)

    ones16 = jnp.ones((EPB, DEG_W), jnp.float32)
    zeros_deg = jnp.zeros((N_ACC, DEG_W), jnp.float32)
    zeros_acc = jnp.zeros((N_ACC, D), jnp.float32)

    deg2 = _deg_call(n_blk, dst_r, ones16, zeros_deg)
    y1, dinv = _pre_call(x, W1, deg2)
    acc1 = _scatter_call(n_blk, y1, src_r, dst_r, zeros_acc)
    h1, y2 = _mid_call(acc1, y1, dinv, b1.reshape(1, D), x, W2)
    acc2 = _scatter_call(n_blk, y2, src_r, dst_r, zeros_acc)
    h2, y3 = _mid_call(acc2, y2, dinv, b2.reshape(1, D), h1, W3)
    acc3 = _scatter_call(n_blk, y3, src_r, dst_r, zeros_acc)
    return _final_call(acc3, y3, dinv, b3.reshape(1, D), h2)


# speedup vs baseline: 6.3789x; 6.3789x over previous
import functools

import jax
import jax.numpy as jnp
from jax import lax
from jax.experimental import pallas as pl
from jax.experimental.pallas import tpu as pltpu
from jax.experimental.pallas import tpu_sc as plsc

N_NODES = 10000
D = 128
NC = 2
NS = 16
NW = NC * NS
EPB = 128
N_ACC = 10240
CHUNK = N_ACC // NS
DEG_W = 128
ROWS = 400
_MESH = dict(core_axis_name="c", subcore_axis_name="s")


def _deg_call(n_blk, dst_r, ones_hbm, zeros_hbm):
    @functools.partial(
        pl.kernel,
        out_type=jax.ShapeDtypeStruct((NC, N_ACC, DEG_W), jnp.float32),
        mesh=plsc.VectorSubcoreMesh(**_MESH),
        scratch_types=[
            pltpu.VMEM((n_blk, EPB), jnp.int32),
            pltpu.VMEM((EPB, DEG_W), jnp.float32),
            pltpu.VMEM_SHARED((N_ACC, DEG_W), jnp.float32),
        ],
    )
    def deg_kernel(dst_hbm, ones_h, zeros_h, deg_hbm, idx_v, ones_v, acc_sh):
        c = lax.axis_index("c")
        s = lax.axis_index("s")
        wid = c * NS + s
        pltpu.sync_copy(zeros_h.at[pl.ds(s * CHUNK, CHUNK)],
                        acc_sh.at[pl.ds(s * CHUNK, CHUNK)])
        pltpu.sync_copy(dst_hbm.at[wid], idx_v)
        pltpu.sync_copy(ones_h, ones_v)
        plsc.subcore_barrier()

        @pl.loop(0, n_blk)
        def _(j):
            pltpu.sync_copy(ones_v, acc_sh.at[idx_v.at[j]], add=True)

        plsc.subcore_barrier()
        pltpu.sync_copy(acc_sh.at[pl.ds(s * CHUNK, CHUNK)],
                        deg_hbm.at[c].at[pl.ds(s * CHUNK, CHUNK)])

    return deg_kernel(dst_r, ones_hbm, zeros_hbm)


def _scatter_call(n_blk, y, ei_r, zeros_hbm):
    @functools.partial(
        pl.kernel,
        out_type=jax.ShapeDtypeStruct((NC, N_ACC, D), jnp.float32),
        mesh=plsc.VectorSubcoreMesh(**_MESH),
        scratch_types=[
            pltpu.VMEM((2, EPB), jnp.int32),
            pltpu.VMEM((2, EPB), jnp.int32),
            pltpu.VMEM((EPB, D), jnp.float32),
            pltpu.VMEM((EPB, D), jnp.float32),
            pltpu.VMEM_SHARED((N_ACC, D), jnp.float32),
            pltpu.SemaphoreType.DMA,
            pltpu.SemaphoreType.DMA,
            pltpu.SemaphoreType.DMA,
            pltpu.SemaphoreType.DMA,
        ],
    )
    def scat_kernel(y_hbm, ei_hbm, zeros_h, out_hbm,
                    ib0, ib1, g0, g1, acc_sh, isem0, isem1, gsem0, gsem1):
        c = lax.axis_index("c")
        s = lax.axis_index("s")
        wid = c * NS + s
        ew = ei_hbm.at[wid]
        pltpu.sync_copy(zeros_h.at[pl.ds(s * CHUNK, CHUNK)],
                        acc_sh.at[pl.ds(s * CHUNK, CHUNK)])
        plsc.subcore_barrier()

        pltpu.sync_copy(ew.at[pl.ds(0, 2)], ib0)
        pltpu.async_copy(y_hbm.at[ib0.at[0]], g0, gsem0)
        pltpu.async_copy(ew.at[pl.ds(2, 2)], ib1, isem1)

        @pl.loop(0, n_blk - 2, step=2)
        def _(j):
            pltpu.make_async_copy(ew.at[pl.ds(2 * j + 2, 2)], ib1, isem1).wait()
            pltpu.async_copy(y_hbm.at[ib1.at[0]], g1, gsem1)
            pltpu.make_async_copy(y_hbm.at[ib0.at[0]], g0, gsem0).wait()
            pltpu.sync_copy(g0, acc_sh.at[ib0.at[1]], add=True)
            pltpu.async_copy(ew.at[pl.ds(2 * j + 4, 2)], ib0, isem0)
            pltpu.make_async_copy(y_hbm.at[ib1.at[0]], g1, gsem1).wait()
            pltpu.sync_copy(g1, acc_sh.at[ib1.at[1]], add=True)
            pltpu.async_copy(ew.at[pl.ds(2 * j + 6, 2)], ib1, isem1)
            pltpu.make_async_copy(ew.at[pl.ds(2 * j + 4, 2)], ib0, isem0).wait()
            pltpu.async_copy(y_hbm.at[ib0.at[0]], g0, gsem0)

        pltpu.make_async_copy(ew.at[pl.ds(2 * n_blk - 2, 2)], ib1, isem1).wait()
        pltpu.async_copy(y_hbm.at[ib1.at[0]], g1, gsem1)
        pltpu.make_async_copy(y_hbm.at[ib0.at[0]], g0, gsem0).wait()
        pltpu.sync_copy(g0, acc_sh.at[ib0.at[1]], add=True)
        pltpu.make_async_copy(y_hbm.at[ib1.at[0]], g1, gsem1).wait()
        pltpu.sync_copy(g1, acc_sh.at[ib1.at[1]], add=True)

        plsc.subcore_barrier()
        pltpu.sync_copy(acc_sh.at[pl.ds(s * CHUNK, CHUNK)],
                        out_hbm.at[c].at[pl.ds(s * CHUNK, CHUNK)])

    return scat_kernel(y, ei_r, zeros_hbm)


def _pre_call(x, W1, deg2):
    def body(x_ref, w_ref, deg_ref, y_ref, dinv_ref):
        d = deg_ref[0, :, 0:1] + deg_ref[1, :, 0:1] + 1.0
        dinv = lax.rsqrt(d)
        xw = jnp.dot(x_ref[...], w_ref[...], preferred_element_type=jnp.float32)
        y_ref[...] = xw * dinv
        dinv_ref[...] = jnp.broadcast_to(dinv, (ROWS, D))

    return pl.pallas_call(
        body,
        grid=(N_NODES // ROWS,),
        in_specs=[
            pl.BlockSpec((ROWS, D), lambda i: (i, 0)),
            pl.BlockSpec((D, D), lambda i: (0, 0)),
            pl.BlockSpec((NC, ROWS, DEG_W), lambda i: (0, i, 0)),
        ],
        out_specs=[
            pl.BlockSpec((ROWS, D), lambda i: (i, 0)),
            pl.BlockSpec((ROWS, D), lambda i: (i, 0)),
        ],
        out_shape=[
            jax.ShapeDtypeStruct((N_NODES, D), jnp.float32),
            jax.ShapeDtypeStruct((N_NODES, D), jnp.float32),
        ],
    )(x, W1, deg2)


def _mid_call(acc, y, dinv, b, h_prev, W_next):
    def body(acc_ref, y_ref, dinv_ref, b_ref, hp_ref, w_ref, h_ref, yn_ref):
        conv = dinv_ref[...] * (acc_ref[0] + acc_ref[1] + y_ref[...]) + b_ref[...]
        h = jnp.maximum(conv + hp_ref[...], 0.0)
        h_ref[...] = h
        hw = jnp.dot(h, w_ref[...], preferred_element_type=jnp.float32)
        yn_ref[...] = hw * dinv_ref[...]

    return pl.pallas_call(
        body,
        grid=(N_NODES // ROWS,),
        in_specs=[
            pl.BlockSpec((NC, ROWS, D), lambda i: (0, i, 0)),
            pl.BlockSpec((ROWS, D), lambda i: (i, 0)),
            pl.BlockSpec((ROWS, D), lambda i: (i, 0)),
            pl.BlockSpec((1, D), lambda i: (0, 0)),
            pl.BlockSpec((ROWS, D), lambda i: (i, 0)),
            pl.BlockSpec((D, D), lambda i: (0, 0)),
        ],
        out_specs=[
            pl.BlockSpec((ROWS, D), lambda i: (i, 0)),
            pl.BlockSpec((ROWS, D), lambda i: (i, 0)),
        ],
        out_shape=[
            jax.ShapeDtypeStruct((N_NODES, D), jnp.float32),
            jax.ShapeDtypeStruct((N_NODES, D), jnp.float32),
        ],
    )(acc, y, dinv, b, h_prev, W_next)


def kernel(x, edge_index, W1, b1, W2, b2, W3, b3):
    ei = edge_index.astype(jnp.int32)
    src, dst = ei[0], ei[1]
    e = src.shape[0]
    stride = NW * EPB
    n_blk = -(-e // stride)
    if n_blk % 2:
        n_blk += 1
    e_pad = n_blk * stride
    pad = e_pad - e
    src_p = jnp.concatenate([src, jnp.zeros((pad,), jnp.int32)])
    dst_p = jnp.concatenate([dst, jnp.full((pad,), N_NODES, jnp.int32)])
    src_r = src_p.reshape(NW, n_blk, EPB)
    dst_r = dst_p.reshape(NW, n_blk, EPB)
    ei_r = jnp.stack([src_r, dst_r], axis=2).reshape(NW, 2 * n_blk, EPB)

    ones16 = jnp.ones((EPB, DEG_W), jnp.float32)
    zeros_deg = jnp.zeros((N_ACC, DEG_W), jnp.float32)
    zeros_acc = jnp.zeros((N_ACC, D), jnp.float32)

    deg2 = _deg_call(n_blk, dst_r, ones16, zeros_deg)
    y1, dinv = _pre_call(x, W1, deg2)

    w_stack = jnp.stack([W2, W3, W3])
    b_stack = jnp.stack([b1, b2, b3]).reshape(3, 1, D)

    def step(carry, wb):
        h_prev, y = carry
        w_next, b = wb
        acc = _scatter_call(n_blk, y, ei_r, zeros_acc)
        h, y_next = _mid_call(acc, y, dinv, b, h_prev, w_next)
        return (h, y_next), None

    (h3, _), _ = lax.scan(step, (x, y1), (w_stack, b_stack))
    return h3


# speedup vs baseline: 7.0710x; 1.1085x over previous
import functools

import jax
import jax.numpy as jnp
from jax import lax
from jax.experimental import pallas as pl
from jax.experimental.pallas import tpu as pltpu
from jax.experimental.pallas import tpu_sc as plsc

N_NODES = 10000
D = 128
NC = 2
NS = 16
NW = NC * NS
EPB = 128
N_ACC = 10240
CHUNK = N_ACC // NS
DEG_W = 128
ROWS = 400
_MESH = dict(core_axis_name="c", subcore_axis_name="s")


def _deg_call(n_blk, dst_r, ones_hbm, zeros_hbm):
    @functools.partial(
        pl.kernel,
        out_type=jax.ShapeDtypeStruct((NC, N_ACC, DEG_W), jnp.float32),
        mesh=plsc.VectorSubcoreMesh(**_MESH),
        scratch_types=[
            pltpu.VMEM((n_blk, EPB), jnp.int32),
            pltpu.VMEM((EPB, DEG_W), jnp.float32),
            pltpu.VMEM_SHARED((N_ACC, DEG_W), jnp.float32),
        ],
    )
    def deg_kernel(dst_hbm, ones_h, zeros_h, deg_hbm, idx_v, ones_v, acc_sh):
        c = lax.axis_index("c")
        s = lax.axis_index("s")
        wid = c * NS + s
        pltpu.sync_copy(zeros_h.at[pl.ds(s * CHUNK, CHUNK)],
                        acc_sh.at[pl.ds(s * CHUNK, CHUNK)])
        pltpu.sync_copy(dst_hbm.at[wid], idx_v)
        pltpu.sync_copy(ones_h, ones_v)
        plsc.subcore_barrier()

        @pl.loop(0, n_blk)
        def _(j):
            pltpu.sync_copy(ones_v, acc_sh.at[idx_v.at[j]], add=True)

        plsc.subcore_barrier()
        pltpu.sync_copy(acc_sh.at[pl.ds(s * CHUNK, CHUNK)],
                        deg_hbm.at[c].at[pl.ds(s * CHUNK, CHUNK)])

    return deg_kernel(dst_r, ones_hbm, zeros_hbm)


def _unpack_block(pk, j, t):
    for k in range(EPB // 16):
        v = pk[j, pl.ds(k * 16, 16)]
        t[0, pl.ds(k * 16, 16)] = lax.bitwise_and(v, jnp.int32(16383))
        t[1, pl.ds(k * 16, 16)] = lax.shift_right_logical(v, jnp.int32(14))


def _scatter_call(n_blk, y, pk_r, zeros_hbm):
    @functools.partial(
        pl.kernel,
        out_type=jax.ShapeDtypeStruct((NC, N_ACC, D), jnp.float32),
        mesh=plsc.VectorSubcoreMesh(**_MESH),
        scratch_types=[
            pltpu.VMEM((n_blk, EPB), jnp.int32),
            pltpu.VMEM((2, EPB), jnp.int32),
            pltpu.VMEM((2, EPB), jnp.int32),
            pltpu.VMEM((EPB, D), jnp.float32),
            pltpu.VMEM((EPB, D), jnp.float32),
            pltpu.VMEM_SHARED((N_ACC, D), jnp.float32),
            pltpu.SemaphoreType.DMA,
            pltpu.SemaphoreType.DMA,
        ],
    )
    def scat_kernel(y_hbm, pk_hbm, zeros_h, out_hbm,
                    pk, ia, ib, g0, g1, acc_sh, gsem0, gsem1):
        c = lax.axis_index("c")
        s = lax.axis_index("s")
        wid = c * NS + s
        pltpu.sync_copy(zeros_h.at[pl.ds(s * CHUNK, CHUNK)],
                        acc_sh.at[pl.ds(s * CHUNK, CHUNK)])
        pltpu.sync_copy(pk_hbm.at[wid], pk)
        plsc.subcore_barrier()

        _unpack_block(pk, 0, ia)
        pltpu.async_copy(y_hbm.at[ia.at[0]], g0, gsem0)
        _unpack_block(pk, 1, ib)
        pltpu.async_copy(y_hbm.at[ib.at[0]], g1, gsem1)

        @pl.loop(0, n_blk - 2, step=2)
        def _(j):
            pltpu.make_async_copy(y_hbm.at[ia.at[0]], g0, gsem0).wait()
            pltpu.sync_copy(g0, acc_sh.at[ia.at[1]], add=True)
            _unpack_block(pk, j + 2, ia)
            pltpu.async_copy(y_hbm.at[ia.at[0]], g0, gsem0)
            pltpu.make_async_copy(y_hbm.at[ib.at[0]], g1, gsem1).wait()
            pltpu.sync_copy(g1, acc_sh.at[ib.at[1]], add=True)
            _unpack_block(pk, j + 3, ib)
            pltpu.async_copy(y_hbm.at[ib.at[0]], g1, gsem1)

        pltpu.make_async_copy(y_hbm.at[ia.at[0]], g0, gsem0).wait()
        pltpu.sync_copy(g0, acc_sh.at[ia.at[1]], add=True)
        pltpu.make_async_copy(y_hbm.at[ib.at[0]], g1, gsem1).wait()
        pltpu.sync_copy(g1, acc_sh.at[ib.at[1]], add=True)

        plsc.subcore_barrier()
        pltpu.sync_copy(acc_sh.at[pl.ds(s * CHUNK, CHUNK)],
                        out_hbm.at[c].at[pl.ds(s * CHUNK, CHUNK)])

    return scat_kernel(y, pk_r, zeros_hbm)


def _pre_call(x, W1, deg2):
    def body(x_ref, w_ref, deg_ref, y_ref, dinv_ref):
        d = deg_ref[0, :, 0:1] + deg_ref[1, :, 0:1] + 1.0
        dinv = lax.rsqrt(d)
        xw = jnp.dot(x_ref[...], w_ref[...], preferred_element_type=jnp.float32)
        y_ref[...] = xw * dinv
        dinv_ref[...] = jnp.broadcast_to(dinv, (ROWS, D))

    return pl.pallas_call(
        body,
        grid=(N_NODES // ROWS,),
        in_specs=[
            pl.BlockSpec((ROWS, D), lambda i: (i, 0)),
            pl.BlockSpec((D, D), lambda i: (0, 0)),
            pl.BlockSpec((NC, ROWS, DEG_W), lambda i: (0, i, 0)),
        ],
        out_specs=[
            pl.BlockSpec((ROWS, D), lambda i: (i, 0)),
            pl.BlockSpec((ROWS, D), lambda i: (i, 0)),
        ],
        out_shape=[
            jax.ShapeDtypeStruct((N_NODES, D), jnp.float32),
            jax.ShapeDtypeStruct((N_NODES, D), jnp.float32),
        ],
    )(x, W1, deg2)


def _mid_call(acc, y, dinv, b, h_prev, W_next):
    def body(acc_ref, y_ref, dinv_ref, b_ref, hp_ref, w_ref, h_ref, yn_ref):
        conv = dinv_ref[...] * (acc_ref[0] + acc_ref[1] + y_ref[...]) + b_ref[...]
        h = jnp.maximum(conv + hp_ref[...], 0.0)
        h_ref[...] = h
        hw = jnp.dot(h, w_ref[...], preferred_element_type=jnp.float32)
        yn_ref[...] = hw * dinv_ref[...]

    return pl.pallas_call(
        body,
        grid=(N_NODES // ROWS,),
        in_specs=[
            pl.BlockSpec((NC, ROWS, D), lambda i: (0, i, 0)),
            pl.BlockSpec((ROWS, D), lambda i: (i, 0)),
            pl.BlockSpec((ROWS, D), lambda i: (i, 0)),
            pl.BlockSpec((1, D), lambda i: (0, 0)),
            pl.BlockSpec((ROWS, D), lambda i: (i, 0)),
            pl.BlockSpec((D, D), lambda i: (0, 0)),
        ],
        out_specs=[
            pl.BlockSpec((ROWS, D), lambda i: (i, 0)),
            pl.BlockSpec((ROWS, D), lambda i: (i, 0)),
        ],
        out_shape=[
            jax.ShapeDtypeStruct((N_NODES, D), jnp.float32),
            jax.ShapeDtypeStruct((N_NODES, D), jnp.float32),
        ],
    )(acc, y, dinv, b, h_prev, W_next)


def kernel(x, edge_index, W1, b1, W2, b2, W3, b3):
    ei = edge_index.astype(jnp.int32)
    src, dst = ei[0], ei[1]
    e = src.shape[0]
    stride = NW * EPB
    n_blk = -(-e // stride)
    if n_blk % 2:
        n_blk += 1
    e_pad = n_blk * stride
    pad = e_pad - e
    src_p = jnp.concatenate([src, jnp.zeros((pad,), jnp.int32)])
    dst_p = jnp.concatenate([dst, jnp.full((pad,), N_NODES, jnp.int32)])
    dst_r = dst_p.reshape(NW, n_blk, EPB)
    pk_r = (dst_p * 16384 + src_p).reshape(NW, n_blk, EPB)

    ones16 = jnp.ones((EPB, DEG_W), jnp.float32)
    zeros_deg = jnp.zeros((N_ACC, DEG_W), jnp.float32)
    zeros_acc = jnp.zeros((N_ACC, D), jnp.float32)

    deg2 = _deg_call(n_blk, dst_r, ones16, zeros_deg)
    y1, dinv = _pre_call(x, W1, deg2)

    w_stack = jnp.stack([W2, W3, W3])
    b_stack = jnp.stack([b1, b2, b3]).reshape(3, 1, D)

    def step(carry, wb):
        h_prev, y = carry
        w_next, b = wb
        acc = _scatter_call(n_blk, y, pk_r, zeros_acc)
        h, y_next = _mid_call(acc, y, dinv, b, h_prev, w_next)
        return (h, y_next), None

    (h3, _), _ = lax.scan(step, (x, y1), (w_stack, b_stack))
    return h3
